# baseline (speedup 1.0000x reference)
import functools

import jax
import jax.numpy as jnp
from jax import lax
from jax.experimental import pallas as pl
from jax.experimental.pallas import tpu as pltpu
from jax.experimental.pallas import tpu_sc as plsc

_VOCAB = 100000
_BASIC_DIM = 300
_MODIF_DIM = 100
_OUT_DIM = _BASIC_DIM + _MODIF_DIM
_BATCH = 16384

_NC = 2
_NS = 16
_NW = _NC * _NS
_B_PER_W = _BATCH // _NW
_CHUNK = 64
_NCHUNK = _B_PER_W // _CHUNK


def _make_kernel():
    mesh = plsc.VectorSubcoreMesh(core_axis_name="c", subcore_axis_name="s")

    @functools.partial(
        pl.kernel,
        mesh=mesh,
        out_type=jax.ShapeDtypeStruct((_BATCH, _OUT_DIM), jnp.float32),
        compiler_params=pltpu.CompilerParams(
            use_tc_tiling_on_sc=False, needs_layout_passes=False),
        scratch_types=[
            pltpu.VMEM((_B_PER_W,), jnp.int32),
            pltpu.VMEM((_CHUNK,), jnp.int32),
            pltpu.VMEM((_CHUNK,), jnp.int32),
            pltpu.VMEM((_CHUNK, 2 * _BASIC_DIM), jnp.float32),
            pltpu.VMEM((_CHUNK, 2 * _MODIF_DIM), jnp.float32),
            pltpu.VMEM((_CHUNK, _OUT_DIM), jnp.float32),
            pltpu.SemaphoreType.DMA,
        ],
    )
    def k(ids_hbm, basicp_hbm, modifp_hbm, out_hbm,
          idx_v, pidx, parv, buf_p, buf_m, buf_c, sem):
        wid = lax.axis_index("s") * _NC + lax.axis_index("c")
        base = wid * _B_PER_W
        pltpu.sync_copy(ids_hbm.at[pl.ds(base, _B_PER_W)], idx_v)
        iota = lax.iota(jnp.int32, 16)

        def do_chunk(c, carry):
            def prep(t, carry2):
                v = idx_v[pl.ds(c * _CHUNK + t * 16, 16)]
                pidx[pl.ds(t * 16, 16)] = v >> 1
                parv[pl.ds(t * 16, 16)] = v & 1
                return carry2

            lax.fori_loop(0, _CHUNK // 16, prep, 0)

            ga = pltpu.async_copy(basicp_hbm.at[pidx], buf_p, sem)
            gb = pltpu.async_copy(modifp_hbm.at[pidx], buf_m, sem)
            ga.wait()
            gb.wait()

            def interleave(r, carry2):
                rr = jnp.full((16,), r, jnp.int32)
                parb = plsc.load_gather(parv, [rr])
                cb = parb * _BASIC_DIM + iota
                cm = parb * _MODIF_DIM + iota
                for j in range(19):
                    off = 16 * j if j < 18 else _BASIC_DIM - 16
                    v = plsc.load_gather(buf_p, [rr, cb + off])
                    buf_c[r, pl.ds(off, 16)] = v
                for j in range(7):
                    off = 16 * j if j < 6 else _MODIF_DIM - 16
                    v = plsc.load_gather(buf_m, [rr, cm + off])
                    buf_c[r, pl.ds(_BASIC_DIM + off, 16)] = v
                return carry2

            lax.fori_loop(0, _CHUNK, interleave, 0)
            pltpu.sync_copy(
                buf_c, out_hbm.at[pl.ds(base + c * _CHUNK, _CHUNK)])
            return carry

        lax.fori_loop(0, _NCHUNK, do_chunk, 0)

    return k


_kernel_call = _make_kernel()


def kernel(word_ids, basic, modif):
    one = lax.optimization_barrier(jnp.float32(1.0))
    basicp = basic.reshape(_VOCAB // 2, 2 * _BASIC_DIM) * one
    modifp = modif.reshape(_VOCAB // 2, 2 * _MODIF_DIM) * one
    return _kernel_call(word_ids.astype(jnp.int32), basicp, modifp)


# speedup vs baseline: 3.9136x; 3.9136x over previous
import functools

import jax
import jax.numpy as jnp
from jax import lax
from jax.experimental import pallas as pl
from jax.experimental.pallas import tpu as pltpu
from jax.experimental.pallas import tpu_sc as plsc

_VOCAB = 100000
_BASIC_DIM = 300
_MODIF_DIM = 100
_OUT_DIM = _BASIC_DIM + _MODIF_DIM
_BATCH = 16384
_ALIGNED = 256
_TAIL = _BASIC_DIM - _ALIGNED

_NC = 2
_NS = 16
_NW = _NC * _NS
_B_PER_W = _BATCH // _NW
_CHUNK = 64
_NCHUNK = _B_PER_W // _CHUNK


def _make_kernel():
    mesh = plsc.VectorSubcoreMesh(core_axis_name="c", subcore_axis_name="s")

    @functools.partial(
        pl.kernel,
        mesh=mesh,
        out_type=jax.ShapeDtypeStruct((_BATCH, _OUT_DIM), jnp.float32),
        compiler_params=pltpu.CompilerParams(
            use_tc_tiling_on_sc=True, needs_layout_passes=False),
        scratch_types=[
            pltpu.VMEM((_CHUNK,), jnp.int32),
            pltpu.VMEM((_CHUNK, _TAIL), jnp.float32),
            pltpu.VMEM((_CHUNK, _MODIF_DIM), jnp.float32),
            pltpu.VMEM((_CHUNK, _OUT_DIM), jnp.float32),
            pltpu.SemaphoreType.DMA,
            pltpu.SemaphoreType.DMA,
        ],
    )
    def k(ids_hbm, basic_hbm, modif_hbm, out_hbm,
          cidx, buf_t, buf_m, buf_c, sem, sem2):
        wid = lax.axis_index("s") * _NC + lax.axis_index("c")
        base = wid * _B_PER_W

        def do_chunk(c, carry):
            pltpu.sync_copy(ids_hbm.at[pl.ds(base + c * _CHUNK, _CHUNK)], cidx)

            ga = pltpu.async_copy(
                basic_hbm.at[cidx, pl.ds(0, _ALIGNED)],
                buf_c.at[:, pl.ds(0, _ALIGNED)], sem)
            handles = []
            for t in range(_CHUNK // 16):
                vec = cidx[pl.ds(t * 16, 16)]
                for j in range(16):
                    r = t * 16 + j
                    wi = vec[j]
                    handles.append(pltpu.async_copy(
                        basic_hbm.at[wi, pl.ds(_ALIGNED, _TAIL)],
                        buf_t.at[r], sem2))
                    handles.append(pltpu.async_copy(
                        modif_hbm.at[wi], buf_m.at[r], sem2))
            ga.wait()
            for h in handles:
                h.wait()

            def assemble(r, carry2):
                for dst, src in ((256, 0), (272, 16), (284, 28)):
                    buf_c[r, pl.ds(dst, 16)] = buf_t[r, pl.ds(src, 16)]
                for j in range(5):
                    buf_c[r, pl.ds(300 + 16 * j, 16)] = buf_m[r, pl.ds(16 * j, 16)]
                buf_c[r, pl.ds(368, 16)] = buf_m[r, pl.ds(68, 16)]
                buf_c[r, pl.ds(384, 16)] = buf_m[r, pl.ds(84, 16)]
                return carry2

            lax.fori_loop(0, _CHUNK, assemble, 0)
            pltpu.sync_copy(
                buf_c, out_hbm.at[pl.ds(base + c * _CHUNK, _CHUNK)])
            return carry

        lax.fori_loop(0, _NCHUNK, do_chunk, 0)

    return k


_kernel_call = _make_kernel()


def kernel(word_ids, basic, modif):
    return _kernel_call(word_ids.astype(jnp.int32), basic, modif)


# speedup vs baseline: 4.0730x; 1.0407x over previous
import functools

import jax
import jax.numpy as jnp
from jax import lax
from jax.experimental import pallas as pl
from jax.experimental.pallas import tpu as pltpu
from jax.experimental.pallas import tpu_sc as plsc

_VOCAB = 100000
_BASIC_DIM = 300
_MODIF_DIM = 100
_OUT_DIM = _BASIC_DIM + _MODIF_DIM
_BATCH = 16384
_ALIGNED = 256
_TAIL = _BASIC_DIM - _ALIGNED

_NC = 2
_NS = 16
_NW = _NC * _NS
_B_PER_W = _BATCH // _NW
_CHUNK = 64
_NCHUNK = _B_PER_W // _CHUNK
_NBUF = 2


def _make_kernel():
    mesh = plsc.VectorSubcoreMesh(core_axis_name="c", subcore_axis_name="s")

    @functools.partial(
        pl.kernel,
        mesh=mesh,
        out_type=jax.ShapeDtypeStruct((_BATCH, _OUT_DIM), jnp.float32),
        compiler_params=pltpu.CompilerParams(
            use_tc_tiling_on_sc=True, needs_layout_passes=False),
        scratch_types=[
            pltpu.VMEM((_B_PER_W,), jnp.int32),
            pltpu.VMEM((_NBUF, _CHUNK, _TAIL), jnp.float32),
            pltpu.VMEM((_NBUF, _CHUNK, _MODIF_DIM), jnp.float32),
            pltpu.VMEM((_NBUF, _CHUNK, _OUT_DIM), jnp.float32),
            pltpu.SemaphoreType.DMA,
            pltpu.SemaphoreType.DMA,
        ],
    )
    def k(ids_hbm, basic_hbm, modif_hbm, out_hbm,
          idx_v, buf_t, buf_m, buf_c, sem, sem2):
        wid = lax.axis_index("s") * _NC + lax.axis_index("c")
        base = wid * _B_PER_W
        pltpu.sync_copy(ids_hbm.at[pl.ds(base, _B_PER_W)], idx_v)

        def fire(c):
            b = c % _NBUF
            pltpu.async_copy(
                basic_hbm.at[idx_v.at[pl.ds(c * _CHUNK, _CHUNK)],
                             pl.ds(0, _ALIGNED)],
                buf_c.at[b, :, pl.ds(0, _ALIGNED)], sem)
            for t in range(_CHUNK // 16):
                vec = idx_v[pl.ds(c * _CHUNK + t * 16, 16)]
                for j in range(16):
                    r = t * 16 + j
                    wi = vec[j]
                    pltpu.async_copy(
                        basic_hbm.at[wi, pl.ds(_ALIGNED, _TAIL)],
                        buf_t.at[b, r], sem2)
                    pltpu.async_copy(modif_hbm.at[wi], buf_m.at[b, r], sem2)

        def drain(c):
            b = c % _NBUF
            pltpu.make_async_copy(
                basic_hbm.at[pl.ds(0, _CHUNK), pl.ds(0, _ALIGNED)],
                buf_c.at[b, :, pl.ds(0, _ALIGNED)], sem).wait()
            for r in range(_CHUNK):
                pltpu.make_async_copy(
                    basic_hbm.at[0, pl.ds(_ALIGNED, _TAIL)],
                    buf_t.at[b, r], sem2).wait()
                pltpu.make_async_copy(
                    modif_hbm.at[0], buf_m.at[b, r], sem2).wait()

        def finish(c):
            b = c % _NBUF
            def assemble(r, carry2):
                for dst, src in ((256, 0), (272, 16), (284, 28)):
                    buf_c[b, r, pl.ds(dst, 16)] = buf_t[b, r, pl.ds(src, 16)]
                for j in range(5):
                    buf_c[b, r, pl.ds(300 + 16 * j, 16)] = (
                        buf_m[b, r, pl.ds(16 * j, 16)])
                buf_c[b, r, pl.ds(368, 16)] = buf_m[b, r, pl.ds(68, 16)]
                buf_c[b, r, pl.ds(384, 16)] = buf_m[b, r, pl.ds(84, 16)]
                return carry2

            lax.fori_loop(0, _CHUNK, assemble, 0)
            pltpu.sync_copy(
                buf_c.at[b], out_hbm.at[pl.ds(base + c * _CHUNK, _CHUNK)])

        fire(0)

        def steady(i, carry):
            drain(i)
            fire(i + 1)
            finish(i)
            return carry

        lax.fori_loop(0, _NCHUNK - 1, steady, 0)
        drain(_NCHUNK - 1)
        finish(_NCHUNK - 1)

    return k


_kernel_call = _make_kernel()


def kernel(word_ids, basic, modif):
    return _kernel_call(word_ids.astype(jnp.int32), basic, modif)
